# baseline (speedup 1.0000x reference)
import functools

import jax
import jax.numpy as jnp
from jax import lax
from jax.experimental import pallas as pl
from jax.experimental.pallas import tpu as pltpu
from jax.experimental.pallas import tpu_sc as plsc

N_GROUP = 4
INPUT_DIM_GROUP = 250000
OUT_DIM = 16
B = 16384
S = 50
E = B * S
NW = 32
EPW = E // NW
GPW = B // NW
IDX_COLS = 128
ROWS_PW = EPW // IDX_COLS
CHUNK = 2560
G_PER_CHUNK = CHUNK // IDX_COLS
N_CHUNKS = EPW // CHUNK
VECS_PER_ROW = IDX_COLS // 16

_mesh = plsc.VectorSubcoreMesh(core_axis_name="c", subcore_axis_name="s")


@functools.partial(
    pl.kernel,
    out_type=jax.ShapeDtypeStruct((E, OUT_DIM), jnp.float32),
    mesh=_mesh,
    scratch_types=[
        pltpu.VMEM((GPW,), jnp.int32),
        pltpu.VMEM((ROWS_PW, IDX_COLS), jnp.int32),
        pltpu.VMEM((CHUNK, OUT_DIM), jnp.float32),
        pltpu.SemaphoreType.DMA,
    ],
    compiler_params=pltpu.CompilerParams(
        needs_layout_passes=False, use_tc_tiling_on_sc=False
    ),
)
def _sc_gather(idx_hbm, grp_hbm, table_hbm, out_hbm, grp_v, idx_v, rows_v, sem):
    nc = lax.axis_size("c")
    wid = lax.axis_index("s") * nc + lax.axis_index("c")

    pltpu.sync_copy(grp_hbm.at[pl.ds(wid * GPW, GPW)], grp_v)
    pltpu.sync_copy(idx_hbm.at[pl.ds(wid * ROWS_PW, ROWS_PW)], idx_v)

    lanes = lax.iota(jnp.int32, 16)

    def remap_row(r, _):
        for q in range(VECS_PER_ROW):
            pos = r * IDX_COLS + q * 16 + lanes
            b = lax.div(pos, jnp.int32(50))
            g = plsc.load_gather(grp_v, [b])
            sl = pl.ds(q * 16, 16)
            idx_v[r, sl] = idx_v[r, sl] + g * INPUT_DIM_GROUP
        return 0

    lax.fori_loop(0, ROWS_PW, remap_row, 0)

    out_base = wid * EPW
    for c in range(N_CHUNKS):
        def fire(j, _):
            row = c * G_PER_CHUNK + j
            pltpu.async_copy(
                table_hbm.at[idx_v.at[row]],
                rows_v.at[pl.ds(j * IDX_COLS, IDX_COLS)],
                sem,
            )
            return 0

        lax.fori_loop(0, G_PER_CHUNK, fire, 0)
        pltpu.make_async_copy(table_hbm.at[pl.ds(0, CHUNK)], rows_v, sem).wait()
        pltpu.sync_copy(rows_v, out_hbm.at[pl.ds(out_base + c * CHUNK, CHUNK)])


def kernel(indices, group, table):
    idx2d = indices.reshape(E // IDX_COLS, IDX_COLS)
    grp = group.reshape(B)
    out = _sc_gather(idx2d, grp, table)
    return out.reshape(B, 1, S, OUT_DIM)
